# Initial kernel scaffold; baseline (speedup 1.0000x reference)
import jax
import jax.numpy as jnp
from jax.experimental import pallas as pl


def kernel(att_ws):
    raise NotImplementedError("write your pallas kernel here")


# speedup vs baseline: 1.0651x; 1.0651x over previous
import functools

import jax
import jax.numpy as jnp
from jax.experimental import pallas as pl
from jax.experimental.pallas import tpu as pltpu

REDUCTION_FACTOR = 1


def _scores_body(x_ref, s_ref):
    x = x_ref[0]
    maxv = jnp.max(x, axis=1, keepdims=True)
    s_ref[0, 0, 0] = jnp.sum(maxv)


def _select_body(s_ref, head_ref, focus_ref, *, num_heads, l_size):
    def body(i, carry):
        m, idx = carry
        v = s_ref[i, 0, 0]
        better = v > m
        return jnp.where(better, v, m), jnp.where(better, i, idx)

    m, idx = jax.lax.fori_loop(
        0, num_heads, body, (jnp.float32(-jnp.inf), jnp.int32(0))
    )
    head_ref[0] = idx
    focus_ref[0] = m / l_size


def _durations_body(head_ref, x_ref, out_ref, *, t_size, num_chunks):
    del head_ref
    i = pl.program_id(0)
    x = x_ref[0]
    maxv = jnp.max(x, axis=1, keepdims=True)
    ti = jax.lax.broadcasted_iota(jnp.int32, x.shape, 1)
    am = jnp.min(jnp.where(x == maxv, ti, t_size), axis=1, keepdims=True)
    part = jnp.sum((am == ti).astype(jnp.int32), axis=0, keepdims=True)

    @pl.when(i == 0)
    def _init():
        out_ref[...] = part

    @pl.when(i > 0)
    def _acc():
        out_ref[...] += part


def kernel(att_ws):
    L = att_ws.shape[-2]
    T = att_ws.shape[-1]
    a = jnp.reshape(att_ws, (-1, L, T))
    H = a.shape[0]

    scores = pl.pallas_call(
        _scores_body,
        grid=(H,),
        in_specs=[pl.BlockSpec((1, L, T), lambda h: (h, 0, 0))],
        out_specs=pl.BlockSpec(
            (1, 1, 1), lambda h: (h, 0, 0), memory_space=pltpu.SMEM
        ),
        out_shape=jax.ShapeDtypeStruct((H, 1, 1), jnp.float32),
        compiler_params=pltpu.CompilerParams(
            dimension_semantics=("parallel",)
        ),
    )(a)

    head, focus = pl.pallas_call(
        functools.partial(_select_body, num_heads=H, l_size=L),
        in_specs=[pl.BlockSpec(memory_space=pltpu.SMEM)],
        out_specs=(
            pl.BlockSpec(memory_space=pltpu.SMEM),
            pl.BlockSpec(memory_space=pltpu.SMEM),
        ),
        out_shape=(
            jax.ShapeDtypeStruct((1,), jnp.int32),
            jax.ShapeDtypeStruct((1,), jnp.float32),
        ),
    )(scores)

    CHUNK = 256
    NCH = L // CHUNK
    grid_spec = pltpu.PrefetchScalarGridSpec(
        num_scalar_prefetch=1,
        grid=(NCH,),
        in_specs=[pl.BlockSpec((1, CHUNK, T), lambda i, h: (h[0], i, 0))],
        out_specs=pl.BlockSpec((1, T), lambda i, h: (0, 0)),
    )
    durations2d = pl.pallas_call(
        functools.partial(_durations_body, t_size=T, num_chunks=NCH),
        grid_spec=grid_spec,
        out_shape=jax.ShapeDtypeStruct((1, T), jnp.int32),
    )(head, a)

    durations = durations2d[0] * REDUCTION_FACTOR
    return (durations, focus[0])


# speedup vs baseline: 1.0765x; 1.0107x over previous
import functools

import jax
import jax.numpy as jnp
from jax.experimental import pallas as pl
from jax.experimental.pallas import tpu as pltpu

REDUCTION_FACTOR = 1


def _scores_body(x_ref, head_ref, focus_ref, acc_ref, *, num_heads, l_size):
    h = pl.program_id(0)
    x = x_ref[0]
    maxv = jnp.max(x, axis=1, keepdims=True)
    acc_ref[h] = jnp.sum(maxv)

    @pl.when(h == num_heads - 1)
    def _select():
        def body(i, carry):
            m, idx = carry
            v = acc_ref[i]
            better = v > m
            return jnp.where(better, v, m), jnp.where(better, i, idx)

        m, idx = jax.lax.fori_loop(
            0, num_heads, body, (jnp.float32(-jnp.inf), jnp.int32(0))
        )
        head_ref[0] = idx
        focus_ref[0] = m / l_size


def _durations_body(head_ref, x_ref, out_ref, *, t_size, num_chunks):
    del head_ref
    i = pl.program_id(0)
    x = x_ref[0]
    maxv = jnp.max(x, axis=1, keepdims=True)
    ti = jax.lax.broadcasted_iota(jnp.int32, x.shape, 1)
    am = jnp.min(jnp.where(x == maxv, ti, t_size), axis=1, keepdims=True)
    part = jnp.sum((am == ti).astype(jnp.int32), axis=0, keepdims=True)

    @pl.when(i == 0)
    def _init():
        out_ref[...] = part

    @pl.when(i > 0)
    def _acc():
        out_ref[...] += part


def kernel(att_ws):
    L = att_ws.shape[-2]
    T = att_ws.shape[-1]
    a = jnp.reshape(att_ws, (-1, L, T))
    H = a.shape[0]

    head, focus = pl.pallas_call(
        functools.partial(_scores_body, num_heads=H, l_size=L),
        grid=(H,),
        in_specs=[pl.BlockSpec((1, L, T), lambda h: (h, 0, 0))],
        out_specs=(
            pl.BlockSpec(memory_space=pltpu.SMEM),
            pl.BlockSpec(memory_space=pltpu.SMEM),
        ),
        out_shape=(
            jax.ShapeDtypeStruct((1,), jnp.int32),
            jax.ShapeDtypeStruct((1,), jnp.float32),
        ),
        scratch_shapes=[pltpu.SMEM((H,), jnp.float32)],
        compiler_params=pltpu.CompilerParams(
            dimension_semantics=("arbitrary",)
        ),
    )(a)

    CHUNK = 256
    NCH = L // CHUNK
    grid_spec = pltpu.PrefetchScalarGridSpec(
        num_scalar_prefetch=1,
        grid=(NCH,),
        in_specs=[pl.BlockSpec((1, CHUNK, T), lambda i, h: (h[0], i, 0))],
        out_specs=pl.BlockSpec((1, T), lambda i, h: (0, 0)),
    )
    durations2d = pl.pallas_call(
        functools.partial(_durations_body, t_size=T, num_chunks=NCH),
        grid_spec=grid_spec,
        out_shape=jax.ShapeDtypeStruct((1, T), jnp.int32),
    )(head, a)

    durations = durations2d[0] * REDUCTION_FACTOR
    return (durations, focus[0])


# speedup vs baseline: 1.0769x; 1.0004x over previous
import functools

import jax
import jax.numpy as jnp
from jax.experimental import pallas as pl
from jax.experimental.pallas import tpu as pltpu

REDUCTION_FACTOR = 1


def _scores_body(
    x_ref, head_ref, focus_ref, acc_ref, *, num_heads, l_size, heads_per_block
):
    g = pl.program_id(0)
    for j in range(heads_per_block):
        x = x_ref[j]
        maxv = jnp.max(x, axis=1, keepdims=True)
        acc_ref[g * heads_per_block + j] = jnp.sum(maxv)

    @pl.when(g == num_heads // heads_per_block - 1)
    def _select():
        def body(i, carry):
            m, idx = carry
            v = acc_ref[i]
            better = v > m
            return jnp.where(better, v, m), jnp.where(better, i, idx)

        m, idx = jax.lax.fori_loop(
            0, num_heads, body, (jnp.float32(-jnp.inf), jnp.int32(0))
        )
        head_ref[0] = idx
        focus_ref[0] = m / l_size


def _durations_body(head_ref, x_ref, out_ref, *, t_size, num_chunks):
    del head_ref
    i = pl.program_id(0)
    x = x_ref[0]
    maxv = jnp.max(x, axis=1, keepdims=True)
    ti = jax.lax.broadcasted_iota(jnp.int32, x.shape, 1)
    am = jnp.min(jnp.where(x == maxv, ti, t_size), axis=1, keepdims=True)
    part = jnp.sum((am == ti).astype(jnp.int32), axis=0, keepdims=True)

    @pl.when(i == 0)
    def _init():
        out_ref[...] = part

    @pl.when(i > 0)
    def _acc():
        out_ref[...] += part


def kernel(att_ws):
    L = att_ws.shape[-2]
    T = att_ws.shape[-1]
    a = jnp.reshape(att_ws, (-1, L, T))
    H = a.shape[0]

    HPB = 2
    head, focus = pl.pallas_call(
        functools.partial(
            _scores_body, num_heads=H, l_size=L, heads_per_block=HPB
        ),
        grid=(H // HPB,),
        in_specs=[pl.BlockSpec((HPB, L, T), lambda h: (h, 0, 0))],
        out_specs=(
            pl.BlockSpec(memory_space=pltpu.SMEM),
            pl.BlockSpec(memory_space=pltpu.SMEM),
        ),
        out_shape=(
            jax.ShapeDtypeStruct((1,), jnp.int32),
            jax.ShapeDtypeStruct((1,), jnp.float32),
        ),
        scratch_shapes=[pltpu.SMEM((H,), jnp.float32)],
        compiler_params=pltpu.CompilerParams(
            dimension_semantics=("arbitrary",)
        ),
    )(a)

    CHUNK = 256
    NCH = L // CHUNK
    grid_spec = pltpu.PrefetchScalarGridSpec(
        num_scalar_prefetch=1,
        grid=(NCH,),
        in_specs=[pl.BlockSpec((1, CHUNK, T), lambda i, h: (h[0], i, 0))],
        out_specs=pl.BlockSpec((1, T), lambda i, h: (0, 0)),
    )
    durations2d = pl.pallas_call(
        functools.partial(_durations_body, t_size=T, num_chunks=NCH),
        grid_spec=grid_spec,
        out_shape=jax.ShapeDtypeStruct((1, T), jnp.int32),
    )(head, a)

    durations = durations2d[0] * REDUCTION_FACTOR
    return (durations, focus[0])


# speedup vs baseline: 1.0840x; 1.0066x over previous
import functools

import jax
import jax.numpy as jnp
from jax.experimental import pallas as pl
from jax.experimental.pallas import tpu as pltpu

REDUCTION_FACTOR = 1


def _scores_body(
    x0_ref, x1_ref, head_ref, focus_ref, acc_ref, *, num_heads, l_size
):
    g = pl.program_id(0)
    half = num_heads // 2
    for j, ref in ((0, x0_ref), (half, x1_ref)):
        x = ref[0]
        maxv = jnp.max(x, axis=1, keepdims=True)
        acc_ref[g + j] = jnp.sum(maxv)

    @pl.when(g == half - 1)
    def _select():
        def body(i, carry):
            m, idx = carry
            v = acc_ref[i]
            better = v > m
            return jnp.where(better, v, m), jnp.where(better, i, idx)

        m, idx = jax.lax.fori_loop(
            0, num_heads, body, (jnp.float32(-jnp.inf), jnp.int32(0))
        )
        head_ref[0] = idx
        focus_ref[0] = m / l_size


def _durations_body(head_ref, x_ref, out_ref, *, t_size, num_chunks):
    del head_ref
    i = pl.program_id(0)
    x = x_ref[0]
    maxv = jnp.max(x, axis=1, keepdims=True)
    ti = jax.lax.broadcasted_iota(jnp.int32, x.shape, 1)
    am = jnp.min(jnp.where(x == maxv, ti, t_size), axis=1, keepdims=True)
    part = jnp.sum((am == ti).astype(jnp.int32), axis=0, keepdims=True)

    @pl.when(i == 0)
    def _init():
        out_ref[...] = part

    @pl.when(i > 0)
    def _acc():
        out_ref[...] += part


def kernel(att_ws):
    L = att_ws.shape[-2]
    T = att_ws.shape[-1]
    a = jnp.reshape(att_ws, (-1, L, T))
    H = a.shape[0]

    half = H // 2
    head, focus = pl.pallas_call(
        functools.partial(_scores_body, num_heads=H, l_size=L),
        grid=(half,),
        in_specs=[
            pl.BlockSpec((1, L, T), lambda h: (h, 0, 0)),
            pl.BlockSpec((1, L, T), lambda h: (h + half, 0, 0)),
        ],
        out_specs=(
            pl.BlockSpec(memory_space=pltpu.SMEM),
            pl.BlockSpec(memory_space=pltpu.SMEM),
        ),
        out_shape=(
            jax.ShapeDtypeStruct((1,), jnp.int32),
            jax.ShapeDtypeStruct((1,), jnp.float32),
        ),
        scratch_shapes=[pltpu.SMEM((H,), jnp.float32)],
        compiler_params=pltpu.CompilerParams(
            dimension_semantics=("arbitrary",)
        ),
    )(a, a)

    CHUNK = 256
    NCH = L // CHUNK
    grid_spec = pltpu.PrefetchScalarGridSpec(
        num_scalar_prefetch=1,
        grid=(NCH,),
        in_specs=[pl.BlockSpec((1, CHUNK, T), lambda i, h: (h[0], i, 0))],
        out_specs=pl.BlockSpec((1, T), lambda i, h: (0, 0)),
    )
    durations2d = pl.pallas_call(
        functools.partial(_durations_body, t_size=T, num_chunks=NCH),
        grid_spec=grid_spec,
        out_shape=jax.ShapeDtypeStruct((1, T), jnp.int32),
    )(head, a)

    durations = durations2d[0] * REDUCTION_FACTOR
    return (durations, focus[0])
